# baseline (speedup 1.0000x reference)
import functools

import jax
import jax.numpy as jnp
from jax import lax
from jax.experimental import pallas as pl
from jax.experimental.pallas import tpu as pltpu
from jax.experimental.pallas import tpu_sc as plsc

_N = 1_000_000
_D = 64
_T = 400
_G = 80
_NCHUNKS = _N // _T
_NW = 32
_L = 16
_NK2 = (_NCHUNKS // _NW + 2) // 2


def _bucket(d):
    x = jnp.maximum(d - 1, 0)
    b = lax.bitcast_convert_type(x.astype(jnp.float32), jnp.int32)
    return jnp.clip((b >> 23) - 126, 0, 10)


def _body(dist_hbm, table_hbm, out_hbm,
          table_stage, table_sh, idx_v, dist_v0, dist_v1, rows_v0, rows_v1,
          sem_t, sem_g, sem_d0, sem_d1, sem_o0, sem_o1):
    cid = lax.axis_index("c")
    sid = lax.axis_index("s")
    wid = sid * 2 + cid

    @pl.when(sid == 0)
    def _():
        pltpu.sync_copy(table_hbm, table_stage)
        pltpu.sync_copy(table_stage, table_sh)

    plsc.subcore_barrier()

    def compute_idx(dist_v):
        def grp_body(j, c2):
            idx_v[pl.ds(j * _L, _L)] = _bucket(dist_v[pl.ds(j * _L, _L)])
            return c2

        lax.fori_loop(0, _T // _L, grp_body, 0)

    def half(i, j, dist_v, rows_v, dist_nv, sem_d, sem_d_next, sem_o):
        k = wid + i * _NW

        @pl.when(k < _NCHUNKS)
        def _():
            kn = k + _NW

            @pl.when(kn < _NCHUNKS)
            def _():
                pltpu.async_copy(
                    dist_hbm.at[pl.ds(kn * _T, _T)], dist_nv, sem_d_next)

            pltpu.make_async_copy(
                dist_hbm.at[pl.ds(k * _T, _T)], dist_v, sem_d).wait()

            compute_idx(dist_v)

            @pl.when(j > 0)
            def _():
                pltpu.make_async_copy(
                    rows_v, out_hbm.at[pl.ds(k * _T, _T)], sem_o).wait()

            handles = []
            for g in range(_T // _G):
                handles.append(pltpu.async_copy(
                    table_sh.at[idx_v.at[pl.ds(g * _G, _G)]],
                    rows_v.at[pl.ds(g * _G, _G)],
                    sem_g,
                ))
            for h in handles:
                h.wait()

            pltpu.async_copy(rows_v, out_hbm.at[pl.ds(k * _T, _T)], sem_o)

    pltpu.async_copy(dist_hbm.at[pl.ds(wid * _T, _T)], dist_v0, sem_d0)

    def iter_body(j, carry):
        half(2 * j, j, dist_v0, rows_v0, dist_v1, sem_d0, sem_d1, sem_o0)
        half(2 * j + 1, j, dist_v1, rows_v1, dist_v0, sem_d1, sem_d0, sem_o1)
        return carry

    lax.fori_loop(0, _NK2, iter_body, 0)

    pltpu.make_async_copy(rows_v0, out_hbm.at[pl.ds(wid * _T, _T)], sem_o0).wait()
    pltpu.make_async_copy(rows_v1, out_hbm.at[pl.ds(wid * _T, _T)], sem_o1).wait()


@functools.cache
def _build():
    mesh = plsc.VectorSubcoreMesh(core_axis_name="c", subcore_axis_name="s")
    return pl.kernel(
        _body,
        mesh=mesh,
        out_type=jax.ShapeDtypeStruct((_N, _D), jnp.float32),
        scratch_types=[
            pltpu.VMEM((11, _D), jnp.float32),
            pltpu.VMEM_SHARED((11, _D), jnp.float32),
            pltpu.VMEM((_T,), jnp.int32),
            pltpu.VMEM((_T,), jnp.int32),
            pltpu.VMEM((_T,), jnp.int32),
            pltpu.VMEM((_T, _D), jnp.float32),
            pltpu.VMEM((_T, _D), jnp.float32),
            pltpu.SemaphoreType.DMA,
            pltpu.SemaphoreType.DMA,
            pltpu.SemaphoreType.DMA,
            pltpu.SemaphoreType.DMA,
            pltpu.SemaphoreType.DMA,
            pltpu.SemaphoreType.DMA,
        ],
        compiler_params=pltpu.CompilerParams(
            needs_layout_passes=False, use_tc_tiling_on_sc=False),
    )


def kernel(distances, table):
    return _build()(distances, table)


# speedup vs baseline: 1.1967x; 1.1967x over previous
import functools

import jax
import jax.numpy as jnp
from jax import lax
from jax.experimental import pallas as pl
from jax.experimental.pallas import tpu as pltpu
from jax.experimental.pallas import tpu_sc as plsc

_N = 1_000_000
_D = 64
_T = 400
_NCHUNKS = _N // _T
_NW = 32
_L = 16
_NK2 = (_NCHUNKS // _NW + 2) // 2


def _bucket(d):
    x = jnp.maximum(d - 1, 0)
    b = lax.bitcast_convert_type(x.astype(jnp.float32), jnp.int32)
    return jnp.clip((b >> 23) - 126, 0, 10)


def _body(dist_hbm, table_hbm, out_hbm,
          table_v, dist_v0, dist_v1, rows_v0, rows_v1,
          sem_d0, sem_d1, sem_o0, sem_o1):
    cid = lax.axis_index("c")
    sid = lax.axis_index("s")
    wid = sid * 2 + cid

    pltpu.sync_copy(table_hbm, table_v)

    cols = [lax.iota(jnp.int32, _L) + q * _L for q in range(_D // _L)]
    lane_consts = [jnp.full((_L,), r, jnp.int32) for r in range(_L)]

    rowbase = lax.iota(jnp.int32, _L) * _D

    def expand(dist_v, rows_v):
        @plsc.parallel_loop(0, _T // _L, 1, unroll=4)
        def grp_body(j):
            iv = _bucket(dist_v[pl.ds(j * _L, _L)]) * _D
            ob = rowbase + j * (_L * _D)
            for r in range(_L):
                tb = iv.at[lane_consts[r]].get(mode="promise_in_bounds")
                sb = ob.at[lane_consts[r]].get(mode="promise_in_bounds")
                for q in range(_D // _L):
                    v = plsc.load_gather(table_v, [tb + cols[q]])
                    plsc.store_scatter(rows_v, [sb + cols[q]], v)

    def half(i, j, dist_v, rows_v, dist_nv, sem_d, sem_d_next, sem_o):
        k = wid + i * _NW

        @pl.when(k < _NCHUNKS)
        def _():
            kn = k + _NW

            @pl.when(kn < _NCHUNKS)
            def _():
                pltpu.async_copy(
                    dist_hbm.at[pl.ds(kn * _T, _T)], dist_nv, sem_d_next)

            pltpu.make_async_copy(
                dist_hbm.at[pl.ds(k * _T, _T)], dist_v, sem_d).wait()

            @pl.when(j > 0)
            def _():
                pltpu.make_async_copy(
                    rows_v, out_hbm.at[pl.ds(k * (_T * _D), _T * _D)], sem_o).wait()

            expand(dist_v, rows_v)
            pltpu.async_copy(rows_v, out_hbm.at[pl.ds(k * (_T * _D), _T * _D)], sem_o)

    pltpu.async_copy(dist_hbm.at[pl.ds(wid * _T, _T)], dist_v0, sem_d0)

    def iter_body(j, carry):
        half(2 * j, j, dist_v0, rows_v0, dist_v1, sem_d0, sem_d1, sem_o0)
        half(2 * j + 1, j, dist_v1, rows_v1, dist_v0, sem_d1, sem_d0, sem_o1)
        return carry

    lax.fori_loop(0, _NK2, iter_body, 0)

    pltpu.make_async_copy(rows_v0, out_hbm.at[pl.ds(wid * (_T * _D), _T * _D)], sem_o0).wait()
    pltpu.make_async_copy(rows_v1, out_hbm.at[pl.ds(wid * (_T * _D), _T * _D)], sem_o1).wait()


@functools.cache
def _build():
    mesh = plsc.VectorSubcoreMesh(core_axis_name="c", subcore_axis_name="s")
    return pl.kernel(
        _body,
        mesh=mesh,
        out_type=jax.ShapeDtypeStruct((_N * _D,), jnp.float32),
        scratch_types=[
            pltpu.VMEM((11 * _D,), jnp.float32),
            pltpu.VMEM((_T,), jnp.int32),
            pltpu.VMEM((_T,), jnp.int32),
            pltpu.VMEM((_T * _D,), jnp.float32),
            pltpu.VMEM((_T * _D,), jnp.float32),
            pltpu.SemaphoreType.DMA,
            pltpu.SemaphoreType.DMA,
            pltpu.SemaphoreType.DMA,
            pltpu.SemaphoreType.DMA,
        ],
        compiler_params=pltpu.CompilerParams(needs_layout_passes=False),
    )


def kernel(distances, table):
    return _build()(distances, table.reshape(-1)).reshape(_N, _D)
